# Initial kernel scaffold; baseline (speedup 1.0000x reference)
import jax
import jax.numpy as jnp
from jax.experimental import pallas as pl


def kernel(x, eic, eid, eit, batch, Wl1, Wr1, b1, Wl2, Wr2, b2, Wl3, Wr3, b3, Wl4, Wr4, b4, Wl5, Wr5, b5, Wm1, bm1, Wm2, bm2, Wo, bo):
    raise NotImplementedError("write your pallas kernel here")


# speedup vs baseline: 9.1920x; 9.1920x over previous
import functools

import jax
import jax.numpy as jnp
from jax import lax
from jax.experimental import pallas as pl
from jax.experimental.pallas import tpu as pltpu
from jax.experimental.pallas import tpu_sc as plsc

N = 10000
E = 320000
D = 128
G = 64

NC = 2
NS = 16
NW = NC * NS

K = 128
C = 80
E_PAD = NW * C * K
ACC_ROWS = 10240
ZR = ACC_ROWS // NS

BN = 400
GRID = N // BN

def _dot(a, b):
    return jnp.dot(a, b, preferred_element_type=jnp.float32)


def _dot_exact(a, b):
    return jnp.dot(a, b, preferred_element_type=jnp.float32,
                   precision=jax.lax.Precision.HIGHEST)


_MESH = plsc.VectorSubcoreMesh(core_axis_name="c", subcore_axis_name="s",
                               num_cores=NC, num_subcores=NS)


@functools.partial(
    pl.kernel,
    out_type=jax.ShapeDtypeStruct((NC, ACC_ROWS, D), jnp.float32),
    mesh=_MESH,
    scratch_types=[
        pltpu.VMEM_SHARED((ACC_ROWS, D), jnp.float32),
        pltpu.VMEM((4, K), jnp.int32),
        pltpu.VMEM((4, K), jnp.int32),
        pltpu.VMEM((K, D), jnp.float32),
        pltpu.VMEM((K, D), jnp.float32),
        [pltpu.SemaphoreType.DMA] * 4,
        [pltpu.SemaphoreType.DMA] * 4,
        [pltpu.SemaphoreType.DMA] * 2,
    ],
)
def _seg_sc(p_hbm, src_hbm, dst_hbm, z_hbm, out_hbm,
            acc, srcb, dstb, rows0, rows1, ssems, dsems, rsems):
    c = lax.axis_index("c")
    s = lax.axis_index("s")
    pltpu.sync_copy(z_hbm, acc.at[pl.ds(s * ZR, ZR)])
    plsc.subcore_barrier()

    rows = (rows0, rows1)

    def issue_idx(j, slot):
        pltpu.async_copy(src_hbm.at[c, s, j], srcb.at[slot], ssems[slot])
        pltpu.async_copy(dst_hbm.at[c, s, j], dstb.at[slot], dsems[slot])

    def wait_idx(slot):
        dummy = src_hbm.at[0, 0, 0]
        pltpu.make_async_copy(dummy, srcb.at[slot], ssems[slot]).wait()
        pltpu.make_async_copy(dummy, dstb.at[slot], dsems[slot]).wait()

    def issue_gather(slot, b):
        pltpu.async_copy(p_hbm.at[srcb.at[slot]], rows[b], rsems[b])

    def wait_gather(b):
        pltpu.make_async_copy(p_hbm.at[pl.ds(0, K)], rows[b],
                              rsems[b]).wait()

    for j in range(4):
        issue_idx(j, j)
    for b in range(2):
        wait_idx(b)
        issue_gather(b, b)

    def body(m, carry):
        for u in range(4):
            j = 4 * m + u
            b = u % 2
            wait_gather(b)
            pltpu.sync_copy(rows[b], acc.at[dstb.at[u]], add=True)

            @pl.when(j + 4 < C)
            def _():
                issue_idx(j + 4, u)

            @pl.when(j + 2 < C)
            def _():
                wait_idx((u + 2) % 4)
                issue_gather((u + 2) % 4, b)
        return carry

    lax.fori_loop(0, C // 4, body, 0)
    plsc.subcore_barrier()
    pltpu.sync_copy(acc.at[pl.ds(s * ZR, ZR)], out_hbm.at[c, pl.ds(s * ZR, ZR)])


@functools.partial(
    pl.kernel,
    out_type=jax.ShapeDtypeStruct((NC, ACC_ROWS, D), jnp.float32),
    mesh=_MESH,
    scratch_types=[
        pltpu.VMEM_SHARED((ACC_ROWS, D), jnp.float32),
        pltpu.VMEM((C, K), jnp.int32),
        pltpu.VMEM((K, D), jnp.float32),
        pltpu.VMEM((K, D), jnp.float32),
    ],
)
def _deg_sc(dst0_hbm, dst1_hbm, onesA_hbm, onesB_hbm, z_hbm, out_hbm,
            acc, dstv, onesA, onesB):
    c = lax.axis_index("c")
    s = lax.axis_index("s")
    pltpu.sync_copy(z_hbm, acc.at[pl.ds(s * ZR, ZR)])
    pltpu.sync_copy(onesA_hbm, onesA)
    pltpu.sync_copy(onesB_hbm, onesB)
    plsc.subcore_barrier()
    for dh, ov in ((dst0_hbm, onesA), (dst1_hbm, onesB)):
        pltpu.sync_copy(dh.at[c, s], dstv)

        def body(j, carry):
            pltpu.sync_copy(ov, acc.at[dstv.at[j]], add=True)
            return carry

        lax.fori_loop(0, C, body, 0)
    plsc.subcore_barrier()
    pltpu.sync_copy(acc.at[pl.ds(s * ZR, ZR)], out_hbm.at[c, pl.ds(s * ZR, ZR)])


def _wspec():
    return pl.BlockSpec((D, D), lambda i: (0, 0))


def _bspec():
    return pl.BlockSpec((1, D), lambda i: (0, 0))


def _degspec():
    return pl.BlockSpec((1, 1, BN, 2), lambda i: (i, 0, 0, 0))


def _agg(mean, a, s0_ref, s1_ref, d0_ref, d1_ref):
    sm = s0_ref[0] + s1_ref[0]
    if mean:
        deg = d0_ref[0, 0, :, a] + d1_ref[0, 0, :, a]
        sm = sm / jnp.maximum(deg, 1.0)[:, None]
    return sm


def _comb_body(mean, a, s0_ref, s1_ref, d0_ref, d1_ref, h_ref, wl_ref, wr_ref,
               b_ref, o_ref):
    sm = _agg(mean, a, s0_ref, s1_ref, d0_ref, d1_ref)
    o_ref[...] = (_dot(sm, wl_ref[...]) + _dot(h_ref[...], wr_ref[...])
                  + b_ref[...])


def _make_comb(mean, a):
    return pl.pallas_call(
        functools.partial(_comb_body, mean, a),
        grid=(GRID,),
        in_specs=[
            pl.BlockSpec((1, BN, D), lambda i: (0, i, 0)),
            pl.BlockSpec((1, BN, D), lambda i: (1, i, 0)),
            _degspec(),
            _degspec(),
            pl.BlockSpec((BN, D), lambda i: (i, 0)),
            _wspec(), _wspec(), _bspec(),
        ],
        out_specs=pl.BlockSpec((BN, D), lambda i: (i, 0)),
        out_shape=jax.ShapeDtypeStruct((N, D), jnp.float32),
    )


_tc_comb_mean = {a: _make_comb(True, a) for a in (0, 1)}
_tc_comb_sum = _make_comb(False, 0)


def _pool_body(s0_ref, s1_ref, d0_ref, d1_ref, h_ref, batch_ref, wl_ref,
               wr_ref, b_ref, wm1_ref, bm1_ref, wm2_ref, bm2_ref, wo_ref,
               bo_ref, out_ref, acc):
    i = pl.program_id(0)
    sm = _agg(True, 0, s0_ref, s1_ref, d0_ref, d1_ref)
    h = (_dot(sm, wl_ref[...]) + _dot(h_ref[...], wr_ref[...]) + b_ref[...])
    bb = batch_ref[0, 0, :]
    gids = lax.broadcasted_iota(jnp.int32, (G, BN), 0)
    oh = (gids == bb[None, :]).astype(jnp.float32)
    contrib = _dot_exact(oh, h)

    @pl.when(i == 0)
    def _():
        acc[...] = contrib

    @pl.when(i > 0)
    def _():
        acc[...] = acc[...] + contrib

    @pl.when(i == GRID - 1)
    def _():
        g = acc[...]
        g = jnp.maximum(_dot(g, wm1_ref[...]) + bm1_ref[...], 0.0)
        g = jnp.maximum(_dot(g, wm2_ref[...]) + bm2_ref[...], 0.0)
        out_ref[...] = _dot(g, wo_ref[...]) + bo_ref[...]


_tc_pool = pl.pallas_call(
    _pool_body,
    grid=(GRID,),
    in_specs=[
        pl.BlockSpec((1, BN, D), lambda i: (0, i, 0)),
        pl.BlockSpec((1, BN, D), lambda i: (1, i, 0)),
        _degspec(),
        _degspec(),
        pl.BlockSpec((BN, D), lambda i: (i, 0)),
        pl.BlockSpec((1, 1, BN), lambda i: (i, 0, 0)),
        _wspec(), _wspec(), _bspec(),
        _wspec(), _bspec(), _wspec(), _bspec(), _wspec(), _bspec(),
    ],
    out_specs=pl.BlockSpec((G, D), lambda i: (0, 0)),
    out_shape=jax.ShapeDtypeStruct((G, D), jnp.float32),
    scratch_shapes=[pltpu.VMEM((G, D), jnp.float32)],
)


def _prep_edges(ei):
    pad = E_PAD - E
    off = jnp.arange(pad, dtype=jnp.int32)
    src = jnp.concatenate([ei[0], off % N])
    dst = jnp.concatenate([ei[1], N + (off % 16)])
    return (src.reshape(NC, NS, C, K), dst.reshape(NC, NS, C, K))


def kernel(x, eic, eid, eit, batch, Wl1, Wr1, b1, Wl2, Wr2, b2, Wl3, Wr3, b3,
           Wl4, Wr4, b4, Wl5, Wr5, b5, Wm1, bm1, Wm2, bm2, Wo, bo):
    src_ic, dst_ic = _prep_edges(eic)
    src_id, dst_id = _prep_edges(eid)
    src_it, dst_it = _prep_edges(eit)

    z_rows = jnp.zeros((ZR, D), jnp.float32)
    lane = lax.iota(jnp.int32, D)
    onesA = jnp.tile((lane == 0).astype(jnp.float32)[None, :], (K, 1))
    onesB = jnp.tile((lane == 1).astype(jnp.float32)[None, :], (K, 1))

    batch_r = batch.reshape(GRID, 1, BN)
    b1r, b2r, b3r, b4r, b5r = (b.reshape(1, D) for b in (b1, b2, b3, b4, b5))
    bm1r, bm2r = bm1.reshape(1, D), bm2.reshape(1, D)
    wo_pad = jnp.pad(Wo, ((0, 0), (0, D - 1)))
    bo_pad = jnp.pad(bo, (0, D - 1)).reshape(1, D)

    degp = _deg_sc(dst_ic, dst_id, onesA, onesB, z_rows)
    degp0 = degp[0, :N, :2].reshape(GRID, 1, BN, 2)
    degp1 = degp[1, :N, :2].reshape(GRID, 1, BN, 2)

    def seg(p, se):
        return _seg_sc(p, se[0], se[1], z_rows)

    ic, idd, it = (src_ic, dst_ic), (src_id, dst_id), (src_it, dst_it)

    h = x
    sp = seg(h, idd)
    h = _tc_comb_mean[1](sp, sp, degp0, degp1, h, Wl1, Wr1, b1r)
    sp = seg(h, ic)
    h = _tc_comb_mean[0](sp, sp, degp0, degp1, h, Wl2, Wr2, b2r)
    sp = seg(h, ic)
    h = _tc_comb_mean[0](sp, sp, degp0, degp1, h, Wl2, Wr2, b2r)
    sp = seg(h, it)
    h = _tc_comb_sum(sp, sp, degp0, degp1, h, Wl3, Wr3, b3r)
    sp = seg(h, idd)
    h = _tc_comb_mean[1](sp, sp, degp0, degp1, h, Wl4, Wr4, b4r)
    sp = seg(h, ic)
    h = _tc_comb_mean[0](sp, sp, degp0, degp1, h, Wl5, Wr5, b5r)
    sp = seg(h, ic)
    out_full = _tc_pool(sp, sp, degp0, degp1, h, batch_r, Wl5, Wr5, b5r,
                        Wm1, bm1r, Wm2, bm2r, wo_pad, bo_pad)
    return out_full[:, :1]
